# baseline (device time: 177568 ns/iter reference)
import jax
import jax.numpy as jnp
from jax import lax
from jax.experimental import pallas as pl
from jax.experimental.pallas import tpu as pltpu

N_DEV = 8
M, K_SHARD, N = 4096, 512, 2048
BLK = M // N_DEV
HN = N // 2
SUB = BLK // 2


def kernel(x, w_mat):
    x = x.astype(jnp.bfloat16)
    w_mat = w_mat.astype(jnp.bfloat16)

    def body(x_ref, w_ref, out_ref, commR_ref, commL_ref, pc_ref, amax_ref,
             qR_ref, qL_ref, rsR_send, rsR_recv, rsL_send, rsL_recv,
             am_send, am_recv, agR_send, agR_recv, agL_send, agL_recv):
        my = lax.axis_index("i")

        def perm(v):
            return jnp.where(v < 4, v, 11 - v)

        my_pos = perm(my)
        right = perm(lax.rem(my_pos + 1, N_DEV))
        left = perm(lax.rem(my_pos + N_DEV - 1, N_DEV))

        def rows(c):
            return pl.ds(c * BLK, BLK)

        def pcR(c):
            return jnp.dot(x_ref[rows(c), :], w_ref[:, :HN],
                           preferred_element_type=jnp.float32)

        def pcL(c):
            return jnp.dot(x_ref[rows(c), :], w_ref[:, HN:],
                           preferred_element_type=jnp.float32)

        def cR(s):
            return lax.rem(my_pos - s + 2 * N_DEV, N_DEV)

        def cL(s):
            return lax.rem(my_pos + s, N_DEV)

        def sub_rows(k):
            return pl.ds(k * SUB, SUB)

        def pcR_sub(c, k):
            return jnp.dot(x_ref[pl.ds(c * BLK + k * SUB, SUB), :],
                           w_ref[:, :HN], preferred_element_type=jnp.float32)

        def pcL_sub(c, k):
            return jnp.dot(x_ref[pl.ds(c * BLK + k * SUB, SUB), :],
                           w_ref[:, HN:], preferred_element_type=jnp.float32)

        commR_ref[0, sub_rows(0)] = pcR_sub(cR(0), 0).astype(jnp.bfloat16)
        commL_ref[0, sub_rows(0)] = pcL_sub(cL(0), 0).astype(jnp.bfloat16)

        barrier_sem = pltpu.get_barrier_semaphore()
        for nbr in [left, right]:
            pl.semaphore_signal(barrier_sem, inc=1, device_id=(nbr,),
                                device_id_type=pl.DeviceIdType.MESH)
        pl.semaphore_wait(barrier_sem, 2)

        def rs_desc(s, k, rightward):
            comm = commR_ref if rightward else commL_ref
            ssem = rsR_send if rightward else rsL_send
            rsem = rsR_recv if rightward else rsL_recv
            return pltpu.make_async_remote_copy(
                src_ref=comm.at[s % 2, sub_rows(k)],
                dst_ref=comm.at[(s + 1) % 2, sub_rows(k)],
                send_sem=ssem.at[s % 2, k],
                recv_sem=rsem.at[(s + 1) % 2, k],
                device_id=(right if rightward else left,),
                device_id_type=pl.DeviceIdType.MESH,
            )

        desc = {(s, k, rw): rs_desc(s, k, rw)
                for s in range(N_DEV - 1) for k in (0, 1)
                for rw in (True, False)}

        desc[(0, 0, True)].start()
        desc[(0, 0, False)].start()
        commR_ref[0, sub_rows(1)] = pcR_sub(cR(0), 1).astype(jnp.bfloat16)
        commL_ref[0, sub_rows(1)] = pcL_sub(cL(0), 1).astype(jnp.bfloat16)
        desc[(0, 1, True)].start()
        desc[(0, 1, False)].start()
        pc_ref[:, :HN] = pcR(cR(1))
        pc_ref[:, HN:] = pcL(cL(1))

        for s in range(N_DEV - 1):
            final = s == N_DEV - 2
            recv_slot = (s + 1) % 2
            for k in (0, 1):
                for rw in (True, False):
                    comm = commR_ref if rw else commL_ref
                    cols = slice(0, HN) if rw else slice(HN, N)
                    d = desc[(s, k, rw)]
                    d.wait_recv()
                    if s >= 1:
                        desc[(s - 1, k, rw)].wait_send()
                    sub_sum = (comm[recv_slot, sub_rows(k)]
                               .astype(jnp.float32)
                               + pc_ref[sub_rows(k), cols])
                    if not final:
                        comm[recv_slot, sub_rows(k)] = (
                            sub_sum.astype(jnp.bfloat16))
                        desc[(s + 1, k, rw)].start()
                    else:
                        pc_ref[sub_rows(k), cols] = jnp.maximum(sub_sum, 0.0)
            if s < N_DEV - 2:
                pc_ref[:, :HN] = pcR(cR(s + 2))
                pc_ref[:, HN:] = pcL(cL(s + 2))
        for k in (0, 1):
            desc[(N_DEV - 2, k, True)].wait_send()
            desc[(N_DEV - 2, k, False)].wait_send()

        mineR = cR(N_DEV - 1)
        mineL = cL(N_DEV - 1)

        amax_ref[pl.ds(my, 1)] = (jnp.zeros((1, 8, 128), jnp.float32)
                                  + jnp.max(pc_ref[...]))
        bcasts = []
        for o in range(1, N_DEV):
            peer = lax.rem(my + o, N_DEV)
            r = pltpu.make_async_remote_copy(
                src_ref=amax_ref.at[my],
                dst_ref=amax_ref.at[my],
                send_sem=am_send.at[o],
                recv_sem=am_recv.at[my],
                device_id=(peer,),
                device_id_type=pl.DeviceIdType.MESH,
            )
            r.start()
            bcasts.append(r)
        for o in range(1, N_DEV):
            peer = lax.rem(my + o, N_DEV)
            wr = pltpu.make_async_remote_copy(
                src_ref=amax_ref.at[my],
                dst_ref=amax_ref.at[peer],
                send_sem=am_send.at[o],
                recv_sem=am_recv.at[peer],
                device_id=(peer,),
                device_id_type=pl.DeviceIdType.MESH,
            )
            wr.wait_recv()
        for r in bcasts:
            r.wait_send()

        gmax = jnp.max(amax_ref[...])

        scale = gmax / 127.0
        inv = jnp.where(gmax > 0.0, 127.0 / gmax, 0.0)
        qR_ref[0] = jnp.clip(jnp.round(pc_ref[:, :HN] * inv),
                             -127.0, 127.0).astype(jnp.int8)
        qL_ref[0] = jnp.clip(jnp.round(pc_ref[:, HN:] * inv),
                             -127.0, 127.0).astype(jnp.int8)

        def ag_desc(s, k, rightward):
            q = qR_ref if rightward else qL_ref
            ssem = agR_send if rightward else agL_send
            rsem = agR_recv if rightward else agL_recv
            return pltpu.make_async_remote_copy(
                src_ref=q.at[s % 2, sub_rows(k)],
                dst_ref=q.at[(s + 1) % 2, sub_rows(k)],
                send_sem=ssem.at[s % 2, k],
                recv_sem=rsem.at[(s + 1) % 2, k],
                device_id=(right if rightward else left,),
                device_id_type=pl.DeviceIdType.MESH,
            )

        agd = {(s, k, rw): ag_desc(s, k, rw)
               for s in range(N_DEV - 1) for k in (0, 1)
               for rw in (True, False)}

        for k in (0, 1):
            agd[(0, k, True)].start()
            agd[(0, k, False)].start()
        out_ref[rows(mineR), :HN] = qR_ref[0].astype(jnp.float32) * scale
        out_ref[rows(mineL), HN:] = qL_ref[0].astype(jnp.float32) * scale

        for s in range(N_DEV - 1):
            recv_slot = (s + 1) % 2
            for k in (0, 1):
                for rw in (True, False):
                    d = agd[(s, k, rw)]
                    d.wait_recv()
                    if s >= 1:
                        agd[(s - 1, k, rw)].wait_send()
                    if s < N_DEV - 2:
                        agd[(s + 1, k, rw)].start()
                    q = qR_ref if rw else qL_ref
                    c = (lax.rem(my_pos - s + 2 * N_DEV, N_DEV) if rw
                         else lax.rem(my_pos + s, N_DEV))
                    cols = slice(0, HN) if rw else slice(HN, N)
                    out_ref[pl.ds(c * BLK + k * SUB, SUB), cols] = (
                        q[recv_slot, sub_rows(k)].astype(jnp.float32) * scale)
        for k in (0, 1):
            agd[(N_DEV - 2, k, True)].wait_send()
            agd[(N_DEV - 2, k, False)].wait_send()

    return pl.pallas_call(
        body,
        out_shape=jax.ShapeDtypeStruct((M, N), jnp.float32),
        in_specs=[pl.BlockSpec(memory_space=pltpu.VMEM),
                  pl.BlockSpec(memory_space=pltpu.VMEM)],
        out_specs=pl.BlockSpec(memory_space=pltpu.VMEM),
        scratch_shapes=[
            pltpu.VMEM((2, BLK, HN), jnp.bfloat16),
            pltpu.VMEM((2, BLK, HN), jnp.bfloat16),
            pltpu.VMEM((BLK, N), jnp.float32),
            pltpu.VMEM((N_DEV, 8, 128), jnp.float32),
            pltpu.VMEM((2, BLK, HN), jnp.int8),
            pltpu.VMEM((2, BLK, HN), jnp.int8),
            pltpu.SemaphoreType.DMA((2, 2)),
            pltpu.SemaphoreType.DMA((2, 2)),
            pltpu.SemaphoreType.DMA((2, 2)),
            pltpu.SemaphoreType.DMA((2, 2)),
            pltpu.SemaphoreType.DMA((N_DEV,)),
            pltpu.SemaphoreType.DMA((N_DEV,)),
            pltpu.SemaphoreType.DMA((2, 2)),
            pltpu.SemaphoreType.DMA((2, 2)),
            pltpu.SemaphoreType.DMA((2, 2)),
            pltpu.SemaphoreType.DMA((2, 2)),
        ],
        compiler_params=pltpu.CompilerParams(
            collective_id=0, vmem_limit_bytes=60 * 1024 * 1024),
    )(x, w_mat)


# device time: 169683 ns/iter; 1.0465x vs baseline; 1.0465x over previous
import jax
import jax.numpy as jnp
from jax import lax
from jax.experimental import pallas as pl
from jax.experimental.pallas import tpu as pltpu

N_DEV = 8
M, K_SHARD, N = 4096, 512, 2048
BLK = M // N_DEV
HN = N // 2
NSUB = 4
SUB = BLK // NSUB


def kernel(x, w_mat):
    def body(x_ref, w_ref, out_ref, commR_ref, commL_ref, pc_ref, amax_ref,
             qR_ref, qL_ref, rsR_send, rsR_recv, rsL_send, rsL_recv,
             am_send, am_recv, agR_send, agR_recv, agL_send, agL_recv):
        my = lax.axis_index("i")

        def perm(v):
            return jnp.where(v < 4, v, 11 - v)

        my_pos = perm(my)
        right = perm(lax.rem(my_pos + 1, N_DEV))
        left = perm(lax.rem(my_pos + N_DEV - 1, N_DEV))

        def rows(c):
            return pl.ds(c * BLK, BLK)

        def pcR(c):
            return jnp.dot(x_ref[rows(c), :], w_ref[:, :HN],
                           preferred_element_type=jnp.float32)

        def pcL(c):
            return jnp.dot(x_ref[rows(c), :], w_ref[:, HN:],
                           preferred_element_type=jnp.float32)

        def cR(s):
            return lax.rem(my_pos - s + 2 * N_DEV, N_DEV)

        def cL(s):
            return lax.rem(my_pos + s, N_DEV)

        def sub_rows(k):
            return pl.ds(k * SUB, SUB)

        def pcR_sub(c, k):
            return jnp.dot(x_ref[pl.ds(c * BLK + k * SUB, SUB), :],
                           w_ref[:, :HN], preferred_element_type=jnp.float32)

        def pcL_sub(c, k):
            return jnp.dot(x_ref[pl.ds(c * BLK + k * SUB, SUB), :],
                           w_ref[:, HN:], preferred_element_type=jnp.float32)

        commR_ref[0, sub_rows(0)] = pcR_sub(cR(0), 0).astype(jnp.bfloat16)
        commL_ref[0, sub_rows(0)] = pcL_sub(cL(0), 0).astype(jnp.bfloat16)

        barrier_sem = pltpu.get_barrier_semaphore()
        for nbr in [left, right]:
            pl.semaphore_signal(barrier_sem, inc=1, device_id=(nbr,),
                                device_id_type=pl.DeviceIdType.MESH)
        pl.semaphore_wait(barrier_sem, 2)

        def rs_desc(s, k, rightward):
            comm = commR_ref if rightward else commL_ref
            ssem = rsR_send if rightward else rsL_send
            rsem = rsR_recv if rightward else rsL_recv
            return pltpu.make_async_remote_copy(
                src_ref=comm.at[s % 2, sub_rows(k)],
                dst_ref=comm.at[(s + 1) % 2, sub_rows(k)],
                send_sem=ssem.at[s % 2, k],
                recv_sem=rsem.at[(s + 1) % 2, k],
                device_id=(right if rightward else left,),
                device_id_type=pl.DeviceIdType.MESH,
            )

        desc = {(s, k, rw): rs_desc(s, k, rw)
                for s in range(N_DEV - 1) for k in range(NSUB)
                for rw in (True, False)}

        desc[(0, 0, True)].start()
        desc[(0, 0, False)].start()
        for k in range(1, NSUB):
            commR_ref[0, sub_rows(k)] = pcR_sub(cR(0), k).astype(jnp.bfloat16)
            commL_ref[0, sub_rows(k)] = pcL_sub(cL(0), k).astype(jnp.bfloat16)
            desc[(0, k, True)].start()
            desc[(0, k, False)].start()
        pc_ref[:, :HN] = pcR(cR(1))
        pc_ref[:, HN:] = pcL(cL(1))

        for s in range(N_DEV - 1):
            final = s == N_DEV - 2
            recv_slot = (s + 1) % 2
            for k in range(NSUB):
                for rw in (True, False):
                    comm = commR_ref if rw else commL_ref
                    cols = slice(0, HN) if rw else slice(HN, N)
                    d = desc[(s, k, rw)]
                    d.wait_recv()
                    if s >= 1:
                        desc[(s - 1, k, rw)].wait_send()
                    sub_sum = (comm[recv_slot, sub_rows(k)]
                               .astype(jnp.float32)
                               + pc_ref[sub_rows(k), cols])
                    if not final:
                        comm[recv_slot, sub_rows(k)] = (
                            sub_sum.astype(jnp.bfloat16))
                        desc[(s + 1, k, rw)].start()
                    else:
                        pc_ref[sub_rows(k), cols] = jnp.maximum(sub_sum, 0.0)
            if s < N_DEV - 2:
                pc_ref[:, :HN] = pcR(cR(s + 2))
                pc_ref[:, HN:] = pcL(cL(s + 2))
        for k in range(NSUB):
            desc[(N_DEV - 2, k, True)].wait_send()
            desc[(N_DEV - 2, k, False)].wait_send()

        mineR = cR(N_DEV - 1)
        mineL = cL(N_DEV - 1)

        amax_ref[pl.ds(my, 1)] = (jnp.zeros((1, 8, 128), jnp.float32)
                                  + jnp.max(pc_ref[...]))
        bcasts = []
        for o in range(1, N_DEV):
            peer = lax.rem(my + o, N_DEV)
            r = pltpu.make_async_remote_copy(
                src_ref=amax_ref.at[my],
                dst_ref=amax_ref.at[my],
                send_sem=am_send.at[o],
                recv_sem=am_recv.at[my],
                device_id=(peer,),
                device_id_type=pl.DeviceIdType.MESH,
            )
            r.start()
            bcasts.append(r)
        for o in range(1, N_DEV):
            peer = lax.rem(my + o, N_DEV)
            wr = pltpu.make_async_remote_copy(
                src_ref=amax_ref.at[my],
                dst_ref=amax_ref.at[peer],
                send_sem=am_send.at[o],
                recv_sem=am_recv.at[peer],
                device_id=(peer,),
                device_id_type=pl.DeviceIdType.MESH,
            )
            wr.wait_recv()
        for r in bcasts:
            r.wait_send()

        gmax = jnp.max(amax_ref[...])

        scale = gmax / 127.0
        inv = jnp.where(gmax > 0.0, 127.0 / gmax, 0.0)
        qR_ref[0] = jnp.clip(jnp.round(pc_ref[:, :HN] * inv),
                             -127.0, 127.0).astype(jnp.int8)
        qL_ref[0] = jnp.clip(jnp.round(pc_ref[:, HN:] * inv),
                             -127.0, 127.0).astype(jnp.int8)

        def ag_desc(s, k, rightward):
            q = qR_ref if rightward else qL_ref
            ssem = agR_send if rightward else agL_send
            rsem = agR_recv if rightward else agL_recv
            return pltpu.make_async_remote_copy(
                src_ref=q.at[s % 2, sub_rows(k)],
                dst_ref=q.at[(s + 1) % 2, sub_rows(k)],
                send_sem=ssem.at[s % 2, k],
                recv_sem=rsem.at[(s + 1) % 2, k],
                device_id=(right if rightward else left,),
                device_id_type=pl.DeviceIdType.MESH,
            )

        agd = {(s, k, rw): ag_desc(s, k, rw)
               for s in range(N_DEV - 1) for k in range(NSUB)
               for rw in (True, False)}

        for k in range(NSUB):
            agd[(0, k, True)].start()
            agd[(0, k, False)].start()
        out_ref[rows(mineR), :HN] = qR_ref[0].astype(jnp.float32) * scale
        out_ref[rows(mineL), HN:] = qL_ref[0].astype(jnp.float32) * scale

        for s in range(N_DEV - 1):
            recv_slot = (s + 1) % 2
            for k in range(NSUB):
                for rw in (True, False):
                    d = agd[(s, k, rw)]
                    d.wait_recv()
                    if s >= 1:
                        agd[(s - 1, k, rw)].wait_send()
                    if s < N_DEV - 2:
                        agd[(s + 1, k, rw)].start()
                    q = qR_ref if rw else qL_ref
                    c = (lax.rem(my_pos - s + 2 * N_DEV, N_DEV) if rw
                         else lax.rem(my_pos + s, N_DEV))
                    cols = slice(0, HN) if rw else slice(HN, N)
                    out_ref[pl.ds(c * BLK + k * SUB, SUB), cols] = (
                        q[recv_slot, sub_rows(k)].astype(jnp.float32) * scale)
        for k in range(NSUB):
            agd[(N_DEV - 2, k, True)].wait_send()
            agd[(N_DEV - 2, k, False)].wait_send()

    return pl.pallas_call(
        body,
        out_shape=jax.ShapeDtypeStruct((M, N), jnp.float32),
        in_specs=[pl.BlockSpec(memory_space=pltpu.VMEM),
                  pl.BlockSpec(memory_space=pltpu.VMEM)],
        out_specs=pl.BlockSpec(memory_space=pltpu.VMEM),
        scratch_shapes=[
            pltpu.VMEM((2, BLK, HN), jnp.bfloat16),
            pltpu.VMEM((2, BLK, HN), jnp.bfloat16),
            pltpu.VMEM((BLK, N), jnp.float32),
            pltpu.VMEM((N_DEV, 8, 128), jnp.float32),
            pltpu.VMEM((2, BLK, HN), jnp.int8),
            pltpu.VMEM((2, BLK, HN), jnp.int8),
            pltpu.SemaphoreType.DMA((2, NSUB)),
            pltpu.SemaphoreType.DMA((2, NSUB)),
            pltpu.SemaphoreType.DMA((2, NSUB)),
            pltpu.SemaphoreType.DMA((2, NSUB)),
            pltpu.SemaphoreType.DMA((N_DEV,)),
            pltpu.SemaphoreType.DMA((N_DEV,)),
            pltpu.SemaphoreType.DMA((2, NSUB)),
            pltpu.SemaphoreType.DMA((2, NSUB)),
            pltpu.SemaphoreType.DMA((2, NSUB)),
            pltpu.SemaphoreType.DMA((2, NSUB)),
        ],
        compiler_params=pltpu.CompilerParams(
            collective_id=0, vmem_limit_bytes=60 * 1024 * 1024),
    )(x, w_mat)


# device time: 153991 ns/iter; 1.1531x vs baseline; 1.1019x over previous
import jax
import jax.numpy as jnp
from jax import lax
from jax.experimental import pallas as pl
from jax.experimental.pallas import tpu as pltpu

N_DEV = 8
M, K_SHARD, N = 4096, 512, 2048
BLK = M // N_DEV
HN = N // 2
NSUB = 4
SUB = BLK // NSUB


def kernel(x, w_mat):
    def body(x_ref, w_ref, out_ref, commR_ref, commL_ref, pc_ref, amax_ref,
             qR_ref, qL_ref, rsR_send, rsR_recv, rsL_send, rsL_recv,
             am_send, am_recv, agR_send, agR_recv, agL_send, agL_recv):
        my = lax.axis_index("i")

        def perm(v):
            return jnp.where(v < 4, v, 11 - v)

        my_pos = perm(my)
        right = perm(lax.rem(my_pos + 1, N_DEV))
        left = perm(lax.rem(my_pos + N_DEV - 1, N_DEV))

        def rows(c):
            return pl.ds(c * BLK, BLK)

        def pcR(c):
            return jnp.dot(x_ref[rows(c), :], w_ref[:, :HN],
                           preferred_element_type=jnp.float32)

        def pcL(c):
            return jnp.dot(x_ref[rows(c), :], w_ref[:, HN:],
                           preferred_element_type=jnp.float32)

        def cR(s):
            return lax.rem(my_pos - s + 2 * N_DEV, N_DEV)

        def cL(s):
            return lax.rem(my_pos + s, N_DEV)

        def sub_rows(k):
            return pl.ds(k * SUB, SUB)

        def pcR_sub(c, k):
            return jnp.dot(x_ref[pl.ds(c * BLK + k * SUB, SUB), :],
                           w_ref[:, :HN], preferred_element_type=jnp.float32)

        def pcL_sub(c, k):
            return jnp.dot(x_ref[pl.ds(c * BLK + k * SUB, SUB), :],
                           w_ref[:, HN:], preferred_element_type=jnp.float32)

        commR_ref[0, sub_rows(0)] = pcR_sub(cR(0), 0).astype(jnp.bfloat16)
        commL_ref[0, sub_rows(0)] = pcL_sub(cL(0), 0).astype(jnp.bfloat16)

        barrier_sem = pltpu.get_barrier_semaphore()
        for nbr in [left, right]:
            pl.semaphore_signal(barrier_sem, inc=1, device_id=(nbr,),
                                device_id_type=pl.DeviceIdType.MESH)
        pl.semaphore_wait(barrier_sem, 2)

        def rs_desc(s, k, rightward):
            comm = commR_ref if rightward else commL_ref
            ssem = rsR_send if rightward else rsL_send
            rsem = rsR_recv if rightward else rsL_recv
            return pltpu.make_async_remote_copy(
                src_ref=comm.at[s % 2, sub_rows(k)],
                dst_ref=comm.at[(s + 1) % 2, sub_rows(k)],
                send_sem=ssem.at[s % 2, k],
                recv_sem=rsem.at[(s + 1) % 2, k],
                device_id=(right if rightward else left,),
                device_id_type=pl.DeviceIdType.MESH,
            )

        desc = {(s, k, rw): rs_desc(s, k, rw)
                for s in range(N_DEV - 1) for k in range(NSUB)
                for rw in (True, False)}

        desc[(0, 0, True)].start()
        desc[(0, 0, False)].start()
        for k in range(1, NSUB):
            commR_ref[0, sub_rows(k)] = pcR_sub(cR(0), k).astype(jnp.bfloat16)
            commL_ref[0, sub_rows(k)] = pcL_sub(cL(0), k).astype(jnp.bfloat16)
            desc[(0, k, True)].start()
            desc[(0, k, False)].start()
        pc_ref[:, :HN] = pcR(cR(1))
        pc_ref[:, HN:] = pcL(cL(1))

        for s in range(N_DEV - 1):
            final = s == N_DEV - 2
            recv_slot = (s + 1) % 2
            for k in range(NSUB):
                for rw in (True, False):
                    comm = commR_ref if rw else commL_ref
                    cols = slice(0, HN) if rw else slice(HN, N)
                    d = desc[(s, k, rw)]
                    d.wait_recv()
                    if s >= 1:
                        desc[(s - 1, k, rw)].wait_send()
                    sub_sum = (comm[recv_slot, sub_rows(k)]
                               .astype(jnp.float32)
                               + pc_ref[sub_rows(k), cols])
                    if not final:
                        comm[recv_slot, sub_rows(k)] = (
                            sub_sum.astype(jnp.bfloat16))
                        desc[(s + 1, k, rw)].start()
                    else:
                        pc_ref[sub_rows(k), cols] = jnp.maximum(sub_sum, 0.0)
            if s < N_DEV - 2:
                pc_ref[:, :HN] = pcR(cR(s + 2))
                pc_ref[:, HN:] = pcL(cL(s + 2))
        for k in range(NSUB):
            desc[(N_DEV - 2, k, True)].wait_send()
            desc[(N_DEV - 2, k, False)].wait_send()

        mineR = cR(N_DEV - 1)
        mineL = cL(N_DEV - 1)

        amax_ref[pl.ds(my, 1)] = (jnp.zeros((1, 8, 128), jnp.float32)
                                  + jnp.max(pc_ref[...]))
        bcasts = []
        for o in range(1, N_DEV):
            peer = lax.rem(my + o, N_DEV)
            r = pltpu.make_async_remote_copy(
                src_ref=amax_ref.at[my],
                dst_ref=amax_ref.at[my],
                send_sem=am_send.at[o],
                recv_sem=am_recv.at[my],
                device_id=(peer,),
                device_id_type=pl.DeviceIdType.MESH,
            )
            r.start()
            bcasts.append(r)
        for o in range(1, N_DEV):
            peer = lax.rem(my + o, N_DEV)
            wr = pltpu.make_async_remote_copy(
                src_ref=amax_ref.at[my],
                dst_ref=amax_ref.at[peer],
                send_sem=am_send.at[o],
                recv_sem=am_recv.at[peer],
                device_id=(peer,),
                device_id_type=pl.DeviceIdType.MESH,
            )
            wr.wait_recv()
        for r in bcasts:
            r.wait_send()

        gmax = jnp.max(amax_ref[...])

        scale = gmax / 127.0
        inv = jnp.where(gmax > 0.0, 127.0 / gmax, 0.0)
        qR_ref[0] = jnp.clip(jnp.round(pc_ref[:, :HN] * inv),
                             -127.0, 127.0).astype(jnp.int8)
        qL_ref[0] = jnp.clip(jnp.round(pc_ref[:, HN:] * inv),
                             -127.0, 127.0).astype(jnp.int8)

        def ag_desc(s, k, rightward):
            q = qR_ref if rightward else qL_ref
            ssem = agR_send if rightward else agL_send
            rsem = agR_recv if rightward else agL_recv
            return pltpu.make_async_remote_copy(
                src_ref=q.at[s % 2, sub_rows(k)],
                dst_ref=q.at[(s + 1) % 2, sub_rows(k)],
                send_sem=ssem.at[s % 2, k],
                recv_sem=rsem.at[(s + 1) % 2, k],
                device_id=(right if rightward else left,),
                device_id_type=pl.DeviceIdType.MESH,
            )

        agd = {(s, k, rw): ag_desc(s, k, rw)
               for s in range(N_DEV - 1) for k in range(NSUB)
               for rw in (True, False)}

        for k in range(NSUB):
            agd[(0, k, True)].start()
            agd[(0, k, False)].start()
        out_ref[rows(mineR), :HN] = (qR_ref[0].astype(jnp.float32)
                                     * scale).astype(jnp.bfloat16)
        out_ref[rows(mineL), HN:] = (qL_ref[0].astype(jnp.float32)
                                     * scale).astype(jnp.bfloat16)

        for s in range(N_DEV - 1):
            recv_slot = (s + 1) % 2
            for k in range(NSUB):
                for rw in (True, False):
                    d = agd[(s, k, rw)]
                    d.wait_recv()
                    if s >= 1:
                        agd[(s - 1, k, rw)].wait_send()
                    if s < N_DEV - 2:
                        agd[(s + 1, k, rw)].start()
                    q = qR_ref if rw else qL_ref
                    c = (lax.rem(my_pos - s + 2 * N_DEV, N_DEV) if rw
                         else lax.rem(my_pos + s, N_DEV))
                    cols = slice(0, HN) if rw else slice(HN, N)
                    out_ref[pl.ds(c * BLK + k * SUB, SUB), cols] = (
                        (q[recv_slot, sub_rows(k)].astype(jnp.float32)
                         * scale).astype(jnp.bfloat16))
        for k in range(NSUB):
            agd[(N_DEV - 2, k, True)].wait_send()
            agd[(N_DEV - 2, k, False)].wait_send()

    return pl.pallas_call(
        body,
        out_shape=jax.ShapeDtypeStruct((M, N), jnp.bfloat16),
        in_specs=[pl.BlockSpec(memory_space=pltpu.VMEM),
                  pl.BlockSpec(memory_space=pltpu.VMEM)],
        out_specs=pl.BlockSpec(memory_space=pltpu.VMEM),
        scratch_shapes=[
            pltpu.VMEM((2, BLK, HN), jnp.bfloat16),
            pltpu.VMEM((2, BLK, HN), jnp.bfloat16),
            pltpu.VMEM((BLK, N), jnp.float32),
            pltpu.VMEM((N_DEV, 8, 128), jnp.float32),
            pltpu.VMEM((2, BLK, HN), jnp.int8),
            pltpu.VMEM((2, BLK, HN), jnp.int8),
            pltpu.SemaphoreType.DMA((2, NSUB)),
            pltpu.SemaphoreType.DMA((2, NSUB)),
            pltpu.SemaphoreType.DMA((2, NSUB)),
            pltpu.SemaphoreType.DMA((2, NSUB)),
            pltpu.SemaphoreType.DMA((N_DEV,)),
            pltpu.SemaphoreType.DMA((N_DEV,)),
            pltpu.SemaphoreType.DMA((2, NSUB)),
            pltpu.SemaphoreType.DMA((2, NSUB)),
            pltpu.SemaphoreType.DMA((2, NSUB)),
            pltpu.SemaphoreType.DMA((2, NSUB)),
        ],
        compiler_params=pltpu.CompilerParams(
            collective_id=0, vmem_limit_bytes=60 * 1024 * 1024),
    )(x, w_mat)


# device time: 150157 ns/iter; 1.1825x vs baseline; 1.0255x over previous
import jax
import jax.numpy as jnp
from jax import lax
from jax.experimental import pallas as pl
from jax.experimental.pallas import tpu as pltpu

N_DEV = 8
M, K_SHARD, N = 4096, 512, 2048
BLK = M // N_DEV
HN = N // 2
NSUB = 4
SUB = BLK // NSUB


def kernel(x, w_mat):
    def body(x_ref, w_ref, out_ref, ostg_ref, cp_sems,
             commR_ref, commL_ref, pc_ref, amax_ref,
             qR_ref, qL_ref, rsR_send, rsR_recv, rsL_send, rsL_recv,
             am_send, am_recv, agR_send, agR_recv, agL_send, agL_recv):
        my = lax.axis_index("i")

        def perm(v):
            return jnp.where(v < 4, v, 11 - v)

        my_pos = perm(my)
        right = perm(lax.rem(my_pos + 1, N_DEV))
        left = perm(lax.rem(my_pos + N_DEV - 1, N_DEV))

        def rows(c):
            return pl.ds(c * BLK, BLK)

        def pcR(c):
            return jnp.dot(x_ref[rows(c), :], w_ref[:, :HN],
                           preferred_element_type=jnp.float32)

        def pcL(c):
            return jnp.dot(x_ref[rows(c), :], w_ref[:, HN:],
                           preferred_element_type=jnp.float32)

        def cR(s):
            return lax.rem(my_pos - s + 2 * N_DEV, N_DEV)

        def cL(s):
            return lax.rem(my_pos + s, N_DEV)

        def sub_rows(k):
            return pl.ds(k * SUB, SUB)

        def pcR_sub(c, k):
            return jnp.dot(x_ref[pl.ds(c * BLK + k * SUB, SUB), :],
                           w_ref[:, :HN], preferred_element_type=jnp.float32)

        def pcL_sub(c, k):
            return jnp.dot(x_ref[pl.ds(c * BLK + k * SUB, SUB), :],
                           w_ref[:, HN:], preferred_element_type=jnp.float32)

        commR_ref[0, sub_rows(0)] = pcR_sub(cR(0), 0).astype(jnp.bfloat16)
        commL_ref[0, sub_rows(0)] = pcL_sub(cL(0), 0).astype(jnp.bfloat16)

        barrier_sem = pltpu.get_barrier_semaphore()
        for nbr in [left, right]:
            pl.semaphore_signal(barrier_sem, inc=1, device_id=(nbr,),
                                device_id_type=pl.DeviceIdType.MESH)
        pl.semaphore_wait(barrier_sem, 2)

        def rs_desc(s, k, rightward):
            comm = commR_ref if rightward else commL_ref
            ssem = rsR_send if rightward else rsL_send
            rsem = rsR_recv if rightward else rsL_recv
            return pltpu.make_async_remote_copy(
                src_ref=comm.at[s % 2, sub_rows(k)],
                dst_ref=comm.at[(s + 1) % 2, sub_rows(k)],
                send_sem=ssem.at[s % 2, k],
                recv_sem=rsem.at[(s + 1) % 2, k],
                device_id=(right if rightward else left,),
                device_id_type=pl.DeviceIdType.MESH,
            )

        desc = {(s, k, rw): rs_desc(s, k, rw)
                for s in range(N_DEV - 1) for k in range(NSUB)
                for rw in (True, False)}

        desc[(0, 0, True)].start()
        desc[(0, 0, False)].start()
        for k in range(1, NSUB):
            commR_ref[0, sub_rows(k)] = pcR_sub(cR(0), k).astype(jnp.bfloat16)
            commL_ref[0, sub_rows(k)] = pcL_sub(cL(0), k).astype(jnp.bfloat16)
            desc[(0, k, True)].start()
            desc[(0, k, False)].start()
        pc_ref[:, :HN] = pcR(cR(1))
        pc_ref[:, HN:] = pcL(cL(1))

        for s in range(N_DEV - 1):
            final = s == N_DEV - 2
            recv_slot = (s + 1) % 2
            for k in range(NSUB):
                for rw in (True, False):
                    comm = commR_ref if rw else commL_ref
                    cols = slice(0, HN) if rw else slice(HN, N)
                    d = desc[(s, k, rw)]
                    d.wait_recv()
                    if s >= 1:
                        desc[(s - 1, k, rw)].wait_send()
                    sub_sum = (comm[recv_slot, sub_rows(k)]
                               .astype(jnp.float32)
                               + pc_ref[sub_rows(k), cols])
                    if not final:
                        comm[recv_slot, sub_rows(k)] = (
                            sub_sum.astype(jnp.bfloat16))
                        desc[(s + 1, k, rw)].start()
                    else:
                        pc_ref[sub_rows(k), cols] = jnp.maximum(sub_sum, 0.0)
            if s < N_DEV - 2:
                pc_ref[:, :HN] = pcR(cR(s + 2))
                pc_ref[:, HN:] = pcL(cL(s + 2))
        for k in range(NSUB):
            desc[(N_DEV - 2, k, True)].wait_send()
            desc[(N_DEV - 2, k, False)].wait_send()

        mineR = cR(N_DEV - 1)
        mineL = cL(N_DEV - 1)

        amax_ref[pl.ds(my, 1)] = (jnp.zeros((1, 8, 128), jnp.float32)
                                  + jnp.max(pc_ref[...]))
        bcasts = []
        for o in range(1, N_DEV):
            peer = lax.rem(my + o, N_DEV)
            r = pltpu.make_async_remote_copy(
                src_ref=amax_ref.at[my],
                dst_ref=amax_ref.at[my],
                send_sem=am_send.at[o],
                recv_sem=am_recv.at[my],
                device_id=(peer,),
                device_id_type=pl.DeviceIdType.MESH,
            )
            r.start()
            bcasts.append(r)
        for o in range(1, N_DEV):
            peer = lax.rem(my + o, N_DEV)
            wr = pltpu.make_async_remote_copy(
                src_ref=amax_ref.at[my],
                dst_ref=amax_ref.at[peer],
                send_sem=am_send.at[o],
                recv_sem=am_recv.at[peer],
                device_id=(peer,),
                device_id_type=pl.DeviceIdType.MESH,
            )
            wr.wait_recv()
        for r in bcasts:
            r.wait_send()

        gmax = jnp.max(amax_ref[...])

        scale = gmax / 127.0
        inv = jnp.where(gmax > 0.0, 127.0 / gmax, 0.0)
        qR_ref[0] = jnp.clip(jnp.round(pc_ref[:, :HN] * inv),
                             -127.0, 127.0).astype(jnp.int8)
        qL_ref[0] = jnp.clip(jnp.round(pc_ref[:, HN:] * inv),
                             -127.0, 127.0).astype(jnp.int8)

        def ag_desc(s, k, rightward):
            q = qR_ref if rightward else qL_ref
            ssem = agR_send if rightward else agL_send
            rsem = agR_recv if rightward else agL_recv
            return pltpu.make_async_remote_copy(
                src_ref=q.at[s % 2, sub_rows(k)],
                dst_ref=q.at[(s + 1) % 2, sub_rows(k)],
                send_sem=ssem.at[s % 2, k],
                recv_sem=rsem.at[(s + 1) % 2, k],
                device_id=(right if rightward else left,),
                device_id_type=pl.DeviceIdType.MESH,
            )

        agd = {(s, k, rw): ag_desc(s, k, rw)
               for s in range(N_DEV - 1) for k in range(NSUB)
               for rw in (True, False)}

        n_stg = 4
        outstanding = [None] * n_stg
        stg_state = [0]

        def store_block(row_start, col_off, nrows, value):
            slot = stg_state[0] % n_stg
            stg_state[0] += 1
            if outstanding[slot] is not None:
                outstanding[slot].wait()
            ostg_ref[slot, pl.ds(0, nrows)] = value
            cp = pltpu.make_async_copy(
                ostg_ref.at[slot, pl.ds(0, nrows)],
                out_ref.at[pl.ds(row_start, nrows),
                           pl.ds(col_off, HN)],
                cp_sems.at[slot],
            )
            cp.start()
            outstanding[slot] = cp

        for k in range(NSUB):
            agd[(0, k, True)].start()
            agd[(0, k, False)].start()
        store_block(mineR * BLK, 0, BLK,
                    (qR_ref[0].astype(jnp.float32)
                     * scale).astype(jnp.bfloat16))
        store_block(mineL * BLK, HN, BLK,
                    (qL_ref[0].astype(jnp.float32)
                     * scale).astype(jnp.bfloat16))

        for s in range(N_DEV - 1):
            recv_slot = (s + 1) % 2
            for k in range(NSUB):
                for rw in (True, False):
                    d = agd[(s, k, rw)]
                    d.wait_recv()
                    if s >= 1:
                        agd[(s - 1, k, rw)].wait_send()
                    if s < N_DEV - 2:
                        agd[(s + 1, k, rw)].start()
                    q = qR_ref if rw else qL_ref
                    c = (lax.rem(my_pos - s + 2 * N_DEV, N_DEV) if rw
                         else lax.rem(my_pos + s, N_DEV))
                    store_block(c * BLK + k * SUB, 0 if rw else HN, SUB,
                                (q[recv_slot, sub_rows(k)]
                                 .astype(jnp.float32)
                                 * scale).astype(jnp.bfloat16))
        for k in range(NSUB):
            agd[(N_DEV - 2, k, True)].wait_send()
            agd[(N_DEV - 2, k, False)].wait_send()
        for cp in outstanding:
            if cp is not None:
                cp.wait()

    return pl.pallas_call(
        body,
        out_shape=jax.ShapeDtypeStruct((M, N), jnp.bfloat16),
        in_specs=[pl.BlockSpec(memory_space=pltpu.VMEM),
                  pl.BlockSpec(memory_space=pltpu.VMEM)],
        out_specs=pl.BlockSpec(memory_space=pl.ANY),
        scratch_shapes=[
            pltpu.VMEM((4, BLK, HN), jnp.bfloat16),
            pltpu.SemaphoreType.DMA((4,)),
            pltpu.VMEM((2, BLK, HN), jnp.bfloat16),
            pltpu.VMEM((2, BLK, HN), jnp.bfloat16),
            pltpu.VMEM((BLK, N), jnp.float32),
            pltpu.VMEM((N_DEV, 8, 128), jnp.float32),
            pltpu.VMEM((2, BLK, HN), jnp.int8),
            pltpu.VMEM((2, BLK, HN), jnp.int8),
            pltpu.SemaphoreType.DMA((2, NSUB)),
            pltpu.SemaphoreType.DMA((2, NSUB)),
            pltpu.SemaphoreType.DMA((2, NSUB)),
            pltpu.SemaphoreType.DMA((2, NSUB)),
            pltpu.SemaphoreType.DMA((N_DEV,)),
            pltpu.SemaphoreType.DMA((N_DEV,)),
            pltpu.SemaphoreType.DMA((2, NSUB)),
            pltpu.SemaphoreType.DMA((2, NSUB)),
            pltpu.SemaphoreType.DMA((2, NSUB)),
            pltpu.SemaphoreType.DMA((2, NSUB)),
        ],
        compiler_params=pltpu.CompilerParams(
            collective_id=0, vmem_limit_bytes=60 * 1024 * 1024),
    )(x, w_mat)


# device time: 150121 ns/iter; 1.1828x vs baseline; 1.0002x over previous
import jax
import jax.numpy as jnp
from jax import lax
from jax.experimental import pallas as pl
from jax.experimental.pallas import tpu as pltpu

N_DEV = 8
M, K_SHARD, N = 4096, 512, 2048
BLK = M // N_DEV
HN = N // 2
NSUB = 4
SUB = BLK // NSUB


def kernel(x, w_mat):
    def body(x_ref, w_ref, out_ref, ostg_ref, cp_sems,
             commR_ref, commL_ref, pc_ref, amax_ref,
             qR_ref, qL_ref, rsR_send, rsR_recv, rsL_send, rsL_recv,
             am_send, am_recv, agR_send, agR_recv, agL_send, agL_recv):
        my = lax.axis_index("i")

        def perm(v):
            return jnp.where(v < 4, v, 11 - v)

        my_pos = perm(my)
        right = perm(lax.rem(my_pos + 1, N_DEV))
        left = perm(lax.rem(my_pos + N_DEV - 1, N_DEV))

        def rows(c):
            return pl.ds(c * BLK, BLK)

        def pcR(c):
            return jnp.dot(x_ref[rows(c), :], w_ref[:, :HN],
                           preferred_element_type=jnp.float32)

        def pcL(c):
            return jnp.dot(x_ref[rows(c), :], w_ref[:, HN:],
                           preferred_element_type=jnp.float32)

        def cR(s):
            return lax.rem(my_pos - s + 2 * N_DEV, N_DEV)

        def cL(s):
            return lax.rem(my_pos + s, N_DEV)

        def sub_rows(k):
            return pl.ds(k * SUB, SUB)

        def pcR_sub(c, k):
            return jnp.dot(x_ref[pl.ds(c * BLK + k * SUB, SUB), :],
                           w_ref[:, :HN], preferred_element_type=jnp.float32)

        def pcL_sub(c, k):
            return jnp.dot(x_ref[pl.ds(c * BLK + k * SUB, SUB), :],
                           w_ref[:, HN:], preferred_element_type=jnp.float32)

        commR_ref[0, sub_rows(0)] = pcR_sub(cR(0), 0).astype(jnp.bfloat16)
        commL_ref[0, sub_rows(0)] = pcL_sub(cL(0), 0).astype(jnp.bfloat16)

        barrier_sem = pltpu.get_barrier_semaphore()
        for nbr in [left, right]:
            pl.semaphore_signal(barrier_sem, inc=1, device_id=(nbr,),
                                device_id_type=pl.DeviceIdType.MESH)
        pl.semaphore_wait(barrier_sem, 2)

        def rs_desc(s, k, rightward):
            comm = commR_ref if rightward else commL_ref
            ssem = rsR_send if rightward else rsL_send
            rsem = rsR_recv if rightward else rsL_recv
            return pltpu.make_async_remote_copy(
                src_ref=comm.at[s % 2, sub_rows(k)],
                dst_ref=comm.at[(s + 1) % 2, sub_rows(k)],
                send_sem=ssem.at[s % 2, k],
                recv_sem=rsem.at[(s + 1) % 2, k],
                device_id=(right if rightward else left,),
                device_id_type=pl.DeviceIdType.MESH,
            )

        desc = {(s, k, rw): rs_desc(s, k, rw)
                for s in range(N_DEV - 1) for k in range(NSUB)
                for rw in (True, False)}

        desc[(0, 0, True)].start()
        desc[(0, 0, False)].start()
        for k in range(1, NSUB):
            commR_ref[0, sub_rows(k)] = pcR_sub(cR(0), k).astype(jnp.bfloat16)
            commL_ref[0, sub_rows(k)] = pcL_sub(cL(0), k).astype(jnp.bfloat16)
            desc[(0, k, True)].start()
            desc[(0, k, False)].start()
        pc_ref[:, :HN] = pcR(cR(1))
        pc_ref[:, HN:] = pcL(cL(1))

        local_amax = jnp.float32(0.0)
        for s in range(N_DEV - 1):
            final = s == N_DEV - 2
            recv_slot = (s + 1) % 2
            for k in range(NSUB):
                for rw in (True, False):
                    comm = commR_ref if rw else commL_ref
                    cols = slice(0, HN) if rw else slice(HN, N)
                    d = desc[(s, k, rw)]
                    d.wait_recv()
                    if s >= 1:
                        desc[(s - 1, k, rw)].wait_send()
                    sub_sum = (comm[recv_slot, sub_rows(k)]
                               .astype(jnp.float32)
                               + pc_ref[sub_rows(k), cols])
                    if not final:
                        comm[recv_slot, sub_rows(k)] = (
                            sub_sum.astype(jnp.bfloat16))
                        desc[(s + 1, k, rw)].start()
                    else:
                        rel = jnp.maximum(sub_sum, 0.0)
                        pc_ref[sub_rows(k), cols] = rel
                        local_amax = jnp.maximum(local_amax, jnp.max(rel))
            if s < N_DEV - 2:
                pc_ref[:, :HN] = pcR(cR(s + 2))
                pc_ref[:, HN:] = pcL(cL(s + 2))
        for k in range(NSUB):
            desc[(N_DEV - 2, k, True)].wait_send()
            desc[(N_DEV - 2, k, False)].wait_send()

        mineR = cR(N_DEV - 1)
        mineL = cL(N_DEV - 1)

        amax_ref[pl.ds(my, 1)] = (jnp.zeros((1, 8, 128), jnp.float32)
                                  + local_amax)
        bcasts = []
        for o in range(1, N_DEV):
            peer = lax.rem(my + o, N_DEV)
            r = pltpu.make_async_remote_copy(
                src_ref=amax_ref.at[my],
                dst_ref=amax_ref.at[my],
                send_sem=am_send.at[o],
                recv_sem=am_recv.at[my],
                device_id=(peer,),
                device_id_type=pl.DeviceIdType.MESH,
            )
            r.start()
            bcasts.append(r)
        for o in range(1, N_DEV):
            peer = lax.rem(my + o, N_DEV)
            wr = pltpu.make_async_remote_copy(
                src_ref=amax_ref.at[my],
                dst_ref=amax_ref.at[peer],
                send_sem=am_send.at[o],
                recv_sem=am_recv.at[peer],
                device_id=(peer,),
                device_id_type=pl.DeviceIdType.MESH,
            )
            wr.wait_recv()
        for r in bcasts:
            r.wait_send()

        gmax = jnp.max(amax_ref[...])

        scale = gmax / 127.0
        inv = jnp.where(gmax > 0.0, 127.0 / gmax, 0.0)
        qR_ref[0] = jnp.clip(jnp.round(pc_ref[:, :HN] * inv),
                             -127.0, 127.0).astype(jnp.int8)
        qL_ref[0] = jnp.clip(jnp.round(pc_ref[:, HN:] * inv),
                             -127.0, 127.0).astype(jnp.int8)

        def ag_desc(s, k, rightward):
            q = qR_ref if rightward else qL_ref
            ssem = agR_send if rightward else agL_send
            rsem = agR_recv if rightward else agL_recv
            return pltpu.make_async_remote_copy(
                src_ref=q.at[s % 2, sub_rows(k)],
                dst_ref=q.at[(s + 1) % 2, sub_rows(k)],
                send_sem=ssem.at[s % 2, k],
                recv_sem=rsem.at[(s + 1) % 2, k],
                device_id=(right if rightward else left,),
                device_id_type=pl.DeviceIdType.MESH,
            )

        agd = {(s, k, rw): ag_desc(s, k, rw)
               for s in range(N_DEV - 1) for k in range(NSUB)
               for rw in (True, False)}

        n_stg = 4
        outstanding = [None] * n_stg
        stg_state = [0]

        def store_block(row_start, col_off, nrows, value):
            slot = stg_state[0] % n_stg
            stg_state[0] += 1
            if outstanding[slot] is not None:
                outstanding[slot].wait()
            ostg_ref[slot, pl.ds(0, nrows)] = value
            cp = pltpu.make_async_copy(
                ostg_ref.at[slot, pl.ds(0, nrows)],
                out_ref.at[pl.ds(row_start, nrows),
                           pl.ds(col_off, HN)],
                cp_sems.at[slot],
            )
            cp.start()
            outstanding[slot] = cp

        for k in range(NSUB):
            agd[(0, k, True)].start()
            agd[(0, k, False)].start()
        store_block(mineR * BLK, 0, BLK,
                    (qR_ref[0].astype(jnp.float32)
                     * scale).astype(jnp.bfloat16))
        store_block(mineL * BLK, HN, BLK,
                    (qL_ref[0].astype(jnp.float32)
                     * scale).astype(jnp.bfloat16))

        for s in range(N_DEV - 1):
            recv_slot = (s + 1) % 2
            for k in range(NSUB):
                for rw in (True, False):
                    d = agd[(s, k, rw)]
                    d.wait_recv()
                    if s >= 1:
                        agd[(s - 1, k, rw)].wait_send()
                    if s < N_DEV - 2:
                        agd[(s + 1, k, rw)].start()
                    q = qR_ref if rw else qL_ref
                    c = (lax.rem(my_pos - s + 2 * N_DEV, N_DEV) if rw
                         else lax.rem(my_pos + s, N_DEV))
                    store_block(c * BLK + k * SUB, 0 if rw else HN, SUB,
                                (q[recv_slot, sub_rows(k)]
                                 .astype(jnp.float32)
                                 * scale).astype(jnp.bfloat16))
        for k in range(NSUB):
            agd[(N_DEV - 2, k, True)].wait_send()
            agd[(N_DEV - 2, k, False)].wait_send()
        for cp in outstanding:
            if cp is not None:
                cp.wait()

    return pl.pallas_call(
        body,
        out_shape=jax.ShapeDtypeStruct((M, N), jnp.bfloat16),
        in_specs=[pl.BlockSpec(memory_space=pltpu.VMEM),
                  pl.BlockSpec(memory_space=pltpu.VMEM)],
        out_specs=pl.BlockSpec(memory_space=pl.ANY),
        scratch_shapes=[
            pltpu.VMEM((4, BLK, HN), jnp.bfloat16),
            pltpu.SemaphoreType.DMA((4,)),
            pltpu.VMEM((2, BLK, HN), jnp.bfloat16),
            pltpu.VMEM((2, BLK, HN), jnp.bfloat16),
            pltpu.VMEM((BLK, N), jnp.float32),
            pltpu.VMEM((N_DEV, 8, 128), jnp.float32),
            pltpu.VMEM((2, BLK, HN), jnp.int8),
            pltpu.VMEM((2, BLK, HN), jnp.int8),
            pltpu.SemaphoreType.DMA((2, NSUB)),
            pltpu.SemaphoreType.DMA((2, NSUB)),
            pltpu.SemaphoreType.DMA((2, NSUB)),
            pltpu.SemaphoreType.DMA((2, NSUB)),
            pltpu.SemaphoreType.DMA((N_DEV,)),
            pltpu.SemaphoreType.DMA((N_DEV,)),
            pltpu.SemaphoreType.DMA((2, NSUB)),
            pltpu.SemaphoreType.DMA((2, NSUB)),
            pltpu.SemaphoreType.DMA((2, NSUB)),
            pltpu.SemaphoreType.DMA((2, NSUB)),
        ],
        compiler_params=pltpu.CompilerParams(
            collective_id=0, vmem_limit_bytes=60 * 1024 * 1024),
    )(x, w_mat)
